# baseline (speedup 1.0000x reference)
import functools
import numpy as np
import jax
import jax.numpy as jnp
from jax import lax
from jax.experimental import pallas as pl
from jax.experimental.pallas import tpu as pltpu
from jax.experimental.pallas import tpu_sc as plsc

_D_MODEL = 2048
_NC, _NS = 2, 16
_NW = _NC * _NS
_C = 16


def _sin_table(seq_len):
    position = jnp.arange(seq_len, dtype=jnp.float32)[:, None]
    div_term = jnp.exp(
        jnp.arange(0, _D_MODEL, 2, dtype=jnp.float32)
        * (-np.log(10000.0) / _D_MODEL)
    )
    ang = position * div_term
    pe = jnp.zeros((seq_len, _D_MODEL), dtype=jnp.float32)
    pe = pe.at[:, 0::2].set(jnp.sin(ang))
    pe = pe.at[:, 1::2].set(jnp.cos(ang))
    return pe


def kernel(x, pe_learn, mix_weight):
    B, S, D = x.shape
    rows_per_w = S // _NW
    chunks = rows_per_w // _C
    CE = _C * D

    x1 = x.reshape(B * S * D)
    sin1 = _sin_table(S).reshape(S * D)
    learn1 = pe_learn.reshape(-1)
    mw16 = jnp.full((16,), mix_weight, dtype=jnp.float32)

    mesh = plsc.VectorSubcoreMesh(core_axis_name="c", subcore_axis_name="s")

    @functools.partial(
        pl.kernel,
        out_type=jax.ShapeDtypeStruct((B * S * D,), jnp.float32),
        mesh=mesh,
        scratch_types=[
            pltpu.VMEM((CE,), jnp.float32),
            pltpu.VMEM((CE,), jnp.float32),
            pltpu.VMEM((16,), jnp.float32),
        ],
    )
    def sc_k(mw_hbm, x_hbm, sin_hbm, learn_hbm, out_hbm, av, bv, mwv):
        wid = lax.axis_index("s") * _NC + lax.axis_index("c")
        pltpu.sync_copy(mw_hbm, mwv)
        w = 1.0 / (1.0 + jnp.exp(-mwv[...]))
        wm = 1.0 - w
        seq0 = wid * rows_per_w

        def chunk_body(ci, carry):
            off = (seq0 + ci * _C) * D
            pltpu.sync_copy(sin_hbm.at[pl.ds(off, CE)], av)
            pltpu.sync_copy(learn_hbm.at[pl.ds(off, CE)], bv)

            def mix_body(i, c1):
                s = i * 64
                for u in range(4):
                    sl = pl.ds(s + u * 16, 16)
                    bv[sl] = w * av[sl] + wm * bv[sl]
                return c1

            lax.fori_loop(0, CE // 64, mix_body, 0)

            def batch_body(b, c1):
                xoff = b * (S * D) + off
                pltpu.sync_copy(x_hbm.at[pl.ds(xoff, CE)], av)

                def add_body(i, c2):
                    s = i * 64
                    for u in range(4):
                        sl = pl.ds(s + u * 16, 16)
                        av[sl] = av[sl] + bv[sl]
                    return c2

                lax.fori_loop(0, CE // 64, add_body, 0)
                pltpu.sync_copy(av, out_hbm.at[pl.ds(xoff, CE)])
                return c1

            lax.fori_loop(0, B, batch_body, 0)
            return carry

        lax.fori_loop(0, chunks, chunk_body, 0)

    out = sc_k(mw16, x1, sin1, learn1)
    return out.reshape(B, S, D)


# speedup vs baseline: 5.7051x; 5.7051x over previous
import numpy as np
import jax
import jax.numpy as jnp
from jax.experimental import pallas as pl
from jax.experimental.pallas import tpu as pltpu

_D_MODEL = 2048
_BS = 256


def _rotation_tables(seq_len):
    D = _D_MODEL
    pairfreq = jnp.exp(
        jnp.arange(0, D, 2, dtype=jnp.float32) * (-np.log(10000.0) / D)
    )
    g = jnp.repeat(pairfreq, 2)[None, :]
    r = jnp.arange(_BS, dtype=jnp.float32)[:, None]
    t_sin, t_cos = jnp.sin(r * g), jnp.cos(r * g)
    nblk = seq_len // _BS
    base = (jnp.arange(nblk, dtype=jnp.float32) * _BS)[:, None]
    sb, cb = jnp.sin(base * g), jnp.cos(base * g)
    even = (jnp.arange(D) % 2 == 0)[None, :]
    p = jnp.where(even, sb, cb).reshape(nblk, 1, D)
    q = jnp.where(even, cb, -sb).reshape(nblk, 1, D)
    return t_sin, t_cos, p, q


def _body(mw_ref, x_ref, learn_ref, tsin_ref, tcos_ref, p_ref, q_ref,
          o_ref, comb_ref):
    b = jax.lax.rem(pl.program_id(0), 4)

    @pl.when(b == 0)
    def _():
        w = jax.nn.sigmoid(mw_ref[0, 0])
        pe_sin = p_ref[0] * tcos_ref[...] + q_ref[0] * tsin_ref[...]
        comb_ref[...] = w * pe_sin + (1.0 - w) * learn_ref[...]

    o_ref[...] = x_ref[...] + comb_ref[...]


def kernel(x, pe_learn, mix_weight):
    B, S, D = x.shape
    nseq = S // _BS
    mw = jnp.asarray(mix_weight, jnp.float32).reshape(1, 1)
    t_sin, t_cos, p, q = _rotation_tables(S)
    x2 = x.reshape(B * S, D)
    out = pl.pallas_call(
        _body,
        grid=(B * nseq,),
        in_specs=[
            pl.BlockSpec(memory_space=pltpu.SMEM),
            pl.BlockSpec((_BS, D), lambda j: ((j % B) * nseq + j // B, 0)),
            pl.BlockSpec((_BS, D), lambda j: (j // B, 0)),
            pl.BlockSpec((_BS, D), lambda j: (0, 0)),
            pl.BlockSpec((_BS, D), lambda j: (0, 0)),
            pl.BlockSpec((1, 1, D), lambda j: (j // B, 0, 0)),
            pl.BlockSpec((1, 1, D), lambda j: (j // B, 0, 0)),
        ],
        out_specs=pl.BlockSpec((_BS, D), lambda j: ((j % B) * nseq + j // B, 0)),
        out_shape=jax.ShapeDtypeStruct((B * S, D), x.dtype),
        scratch_shapes=[pltpu.VMEM((_BS, D), jnp.float32)],
        compiler_params=pltpu.CompilerParams(
            dimension_semantics=("arbitrary",),
        ),
    )(mw, x2, pe_learn, t_sin, t_cos, p, q)
    return out.reshape(B, S, D)


# speedup vs baseline: 7.2999x; 1.2795x over previous
import numpy as np
import jax
import jax.numpy as jnp
from jax.experimental import pallas as pl
from jax.experimental.pallas import tpu as pltpu

_D_MODEL = 2048
_BS = 256


def _rotation_tables(seq_len):
    D = _D_MODEL
    pairfreq = jnp.exp(
        jnp.arange(0, D, 2, dtype=jnp.float32) * (-np.log(10000.0) / D)
    )
    g = jnp.repeat(pairfreq, 2)[None, :]
    r = jnp.arange(_BS, dtype=jnp.float32)[:, None]
    t_sin, t_cos = jnp.sin(r * g), jnp.cos(r * g)
    nblk = seq_len // _BS
    base = (jnp.arange(nblk, dtype=jnp.float32) * _BS)[:, None]
    sb, cb = jnp.sin(base * g), jnp.cos(base * g)
    even = (jnp.arange(D) % 2 == 0)[None, :]
    p = jnp.where(even, sb, cb).reshape(nblk, 1, D)
    q = jnp.where(even, cb, -sb).reshape(nblk, 1, D)
    return t_sin, t_cos, p, q


def _body(mw_ref, x_ref, learn_ref, tsin_ref, tcos_ref, p_ref, q_ref, o_ref):
    w = jax.nn.sigmoid(mw_ref[0, 0])
    pe_sin = p_ref[0] * tcos_ref[...] + q_ref[0] * tsin_ref[...]
    comb = w * pe_sin + (1.0 - w) * learn_ref[...]
    for b in range(x_ref.shape[0]):
        o_ref[b] = x_ref[b] + comb


def kernel(x, pe_learn, mix_weight):
    B, S, D = x.shape
    mw = jnp.asarray(mix_weight, jnp.float32).reshape(1, 1)
    t_sin, t_cos, p, q = _rotation_tables(S)
    return pl.pallas_call(
        _body,
        grid=(S // _BS,),
        in_specs=[
            pl.BlockSpec(memory_space=pltpu.SMEM),
            pl.BlockSpec((B, _BS, D), lambda i: (0, i, 0)),
            pl.BlockSpec((_BS, D), lambda i: (i, 0)),
            pl.BlockSpec((_BS, D), lambda i: (0, 0)),
            pl.BlockSpec((_BS, D), lambda i: (0, 0)),
            pl.BlockSpec((1, 1, D), lambda i: (i, 0, 0)),
            pl.BlockSpec((1, 1, D), lambda i: (i, 0, 0)),
        ],
        out_specs=pl.BlockSpec((B, _BS, D), lambda i: (0, i, 0)),
        out_shape=jax.ShapeDtypeStruct((B, S, D), x.dtype),
        compiler_params=pltpu.CompilerParams(
            dimension_semantics=("parallel",),
        ),
    )(mw, x, pe_learn, t_sin, t_cos, p, q)


# speedup vs baseline: 7.3356x; 1.0049x over previous
import numpy as np
import jax
import jax.numpy as jnp
from jax.experimental import pallas as pl
from jax.experimental.pallas import tpu as pltpu

_D_MODEL = 2048
_BS = 256


def _rotation_tables(seq_len):
    D = _D_MODEL
    pairfreq = jnp.exp(
        jnp.arange(0, D, 2, dtype=jnp.float32) * (-np.log(10000.0) / D)
    )
    g = jnp.repeat(pairfreq, 2)[None, :]
    r = jnp.arange(_BS, dtype=jnp.float32)[:, None]
    t_sin, t_cos = jnp.sin(r * g), jnp.cos(r * g)
    nblk = seq_len // _BS
    base = (jnp.arange(nblk, dtype=jnp.float32) * _BS)[:, None]
    sb, cb = jnp.sin(base * g), jnp.cos(base * g)
    even = (jnp.arange(D) % 2 == 0)[None, :]
    p = jnp.where(even, sb, cb).reshape(nblk, 1, D)
    q = jnp.where(even, cb, -sb).reshape(nblk, 1, D)
    return t_sin, t_cos, p, q


def _body(mw_ref, x_ref, learn_ref, tsin_ref, tcos_ref, p_ref, q_ref, o_ref):
    w = jax.nn.sigmoid(mw_ref[0, 0])
    p_row, q_row = p_ref[0], q_ref[0]
    for r in range(0, _BS, 8):
        sl = pl.ds(r, 8)
        pe_sin = p_row * tcos_ref[sl, :] + q_row * tsin_ref[sl, :]
        comb = w * pe_sin + (1.0 - w) * learn_ref[sl, :]
        for b in range(x_ref.shape[0]):
            o_ref[b, sl, :] = x_ref[b, sl, :] + comb


def kernel(x, pe_learn, mix_weight):
    B, S, D = x.shape
    mw = jnp.asarray(mix_weight, jnp.float32).reshape(1, 1)
    t_sin, t_cos, p, q = _rotation_tables(S)
    return pl.pallas_call(
        _body,
        grid=(S // _BS,),
        in_specs=[
            pl.BlockSpec(memory_space=pltpu.SMEM),
            pl.BlockSpec((B, _BS, D), lambda i: (0, i, 0)),
            pl.BlockSpec((_BS, D), lambda i: (i, 0)),
            pl.BlockSpec((_BS, D), lambda i: (0, 0)),
            pl.BlockSpec((_BS, D), lambda i: (0, 0)),
            pl.BlockSpec((1, 1, D), lambda i: (i, 0, 0)),
            pl.BlockSpec((1, 1, D), lambda i: (i, 0, 0)),
        ],
        out_specs=pl.BlockSpec((B, _BS, D), lambda i: (0, i, 0)),
        out_shape=jax.ShapeDtypeStruct((B, S, D), x.dtype),
        compiler_params=pltpu.CompilerParams(
            dimension_semantics=("parallel",),
        ),
    )(mw, x, pe_learn, t_sin, t_cos, p, q)
